# Initial kernel scaffold; baseline (speedup 1.0000x reference)
import jax
import jax.numpy as jnp
from jax.experimental import pallas as pl


def kernel(x, edge_index, W1, b1, W2, b2):
    raise NotImplementedError("write your pallas kernel here")


# speedup vs baseline: 5.0678x; 5.0678x over previous
import functools

import jax
import jax.numpy as jnp
from jax import lax
from jax.experimental import pallas as pl
from jax.experimental.pallas import tpu as pltpu
from jax.experimental.pallas import tpu_sc as plsc

LANES = 128
NC = 2
NS = 16
NW = NC * NS
CHUNK = 128
NBUF = 2
SIB = 4


def _sc_mesh():
    return plsc.VectorSubcoreMesh(
        core_axis_name="c", subcore_axis_name="s",
        num_cores=NC, num_subcores=NS)


_SC_PARAMS = pltpu.CompilerParams(needs_layout_passes=False)


def _degree_sc(src2, dst2, zflat, n_pad):
    epw = src2.shape[1]
    nsteps = epw // 16

    @functools.partial(
        pl.kernel,
        out_type=jax.ShapeDtypeStruct((NW, 2, n_pad), jnp.float32),
        mesh=_sc_mesh(),
        scratch_types=[
            pltpu.VMEM((epw,), jnp.int32),
            pltpu.VMEM((epw,), jnp.int32),
            pltpu.VMEM((n_pad,), jnp.float32),
            pltpu.VMEM((n_pad,), jnp.float32),
        ],
        compiler_params=_SC_PARAMS,
    )
    def deg_kernel(src_hbm, dst_hbm, zflat_hbm, out_hbm,
                   se_v, de_v, hist_o, hist_i):
        c = lax.axis_index("c")
        s = lax.axis_index("s")
        wid = c * NS + s
        pltpu.sync_copy(src_hbm.at[wid], se_v)
        pltpu.sync_copy(dst_hbm.at[wid], de_v)
        pltpu.sync_copy(zflat_hbm, hist_o)
        pltpu.sync_copy(zflat_hbm, hist_i)

        ones = jnp.full((16,), 1.0, dtype=jnp.float32)

        def step(i, carry):
            off = i * 16
            vs = se_v[pl.ds(off, 16)]
            plsc.addupdate_scatter(hist_o, [vs], ones)
            vd = de_v[pl.ds(off, 16)]
            plsc.addupdate_scatter(hist_i, [vd], ones)
            return carry

        lax.fori_loop(0, nsteps, step, 0)

        pltpu.sync_copy(hist_o, out_hbm.at[wid, 0])
        pltpu.sync_copy(hist_i, out_hbm.at[wid, 1])

    return deg_kernel(src2, dst2, zflat)


def _scatter_sc(h, edges3, zeros_big, n_pad):
    k_chunks = edges3.shape[1]
    rpt = n_pad // NS

    @functools.partial(
        pl.kernel,
        out_type=jax.ShapeDtypeStruct((NC, n_pad, LANES), jnp.float32),
        mesh=_sc_mesh(),
        scratch_types=[
            pltpu.VMEM((SIB, 2, CHUNK), jnp.int32),
            pltpu.VMEM((NBUF, CHUNK, LANES), jnp.float32),
            pltpu.VMEM_SHARED((n_pad, LANES), jnp.float32),
            pltpu.SemaphoreType.DMA((SIB,)),
            pltpu.SemaphoreType.DMA((NBUF,)),
        ],
        compiler_params=_SC_PARAMS,
    )
    def scat_kernel(h_hbm, edges_hbm, zeros_hbm, out_hbm,
                    idx_v, rows_v, acc_sh, isems, gsems):
        c = lax.axis_index("c")
        s = lax.axis_index("s")
        wid = c * NS + s

        def idx_fetch(j, b):
            pltpu.async_copy(edges_hbm.at[wid, j], idx_v.at[b], isems.at[b])

        def idx_wait(j, b):
            pltpu.make_async_copy(
                edges_hbm.at[wid, j], idx_v.at[b], isems.at[b]).wait()

        def gather(j, b, rb):
            pltpu.async_copy(h_hbm.at[idx_v.at[b, 0]], rows_v.at[rb],
                             gsems.at[rb])

        def gather_wait(j, b, rb):
            pltpu.make_async_copy(h_hbm.at[idx_v.at[b, 0]], rows_v.at[rb],
                                  gsems.at[rb]).wait()

        def scat(j, b, rb):
            pltpu.sync_copy(rows_v.at[rb], acc_sh.at[idx_v.at[b, 1]],
                            add=True)

        for p in range(SIB):
            idx_fetch(p, p)
        pltpu.sync_copy(zeros_hbm.at[pl.ds(s * rpt, rpt)],
                        acc_sh.at[pl.ds(s * rpt, rpt)])
        plsc.subcore_barrier()

        idx_wait(0, 0)
        gather(0, 0, 0)

        def step(j, carry):
            ib = lax.rem(j, SIB)
            pb = lax.rem(j - 1, SIB)
            rb = lax.rem(j, NBUF)
            prb = lax.rem(j - 1, NBUF)
            idx_wait(j, ib)
            gather(j, ib, rb)
            gather_wait(j - 1, pb, prb)
            scat(j - 1, pb, prb)
            @pl.when(j - 1 + SIB < k_chunks)
            def _():
                idx_fetch(j - 1 + SIB, pb)
            return carry

        lax.fori_loop(1, k_chunks, step, 0, unroll=2)

        jl = k_chunks - 1
        lb = lax.rem(jl, SIB)
        lrb = lax.rem(jl, NBUF)
        gather_wait(jl, lb, lrb)
        scat(jl, lb, lrb)

        plsc.subcore_barrier()
        pltpu.sync_copy(acc_sh.at[pl.ds(s * rpt, rpt)],
                        out_hbm.at[c, pl.ds(s * rpt, rpt)])

    return scat_kernel(h, edges3, zeros_big)


def _tc_norms(deg_raw, nrows):
    def body(deg_ref, norm_ref):
        deg = jnp.sum(deg_ref[...], axis=0)
        norm_ref[...] = jnp.where(deg > 0, lax.rsqrt(deg), 0.0)

    return pl.pallas_call(
        body,
        out_shape=jax.ShapeDtypeStruct((2, nrows, LANES), jnp.float32),
    )(deg_raw)


def _tc_scale(x_pad, nsrc_col, n_pad):
    blk = 1024

    def body(x_ref, ns_ref, h0_ref):
        h0_ref[...] = x_ref[...] * ns_ref[...]

    return pl.pallas_call(
        body,
        grid=(n_pad // blk,),
        in_specs=[
            pl.BlockSpec((blk, LANES), lambda i: (i, 0)),
            pl.BlockSpec((blk, 1), lambda i: (i, 0)),
        ],
        out_specs=pl.BlockSpec((blk, LANES), lambda i: (i, 0)),
        out_shape=jax.ShapeDtypeStruct((n_pad, LANES), jnp.float32),
    )(x_pad, nsrc_col)


def _tc_mid(agg1, ndst, nsrc, W1, b1_2d, W2, n_pad):
    blk = 1024
    d_in, d_hid = W1.shape
    d_out = W2.shape[1]

    def body(p_ref, nd_ref, ns_ref, w1_ref, b1_ref, w2_ref, g_ref):
        a = (p_ref[0] + p_ref[1]) * nd_ref[...]
        h = jnp.dot(a, w1_ref[...], preferred_element_type=jnp.float32)
        h = jnp.maximum(h + b1_ref[...], 0.0)
        g_ref[...] = jnp.dot(h * ns_ref[...], w2_ref[...],
                             preferred_element_type=jnp.float32)

    return pl.pallas_call(
        body,
        grid=(n_pad // blk,),
        in_specs=[
            pl.BlockSpec((NC, blk, d_in), lambda i: (0, i, 0)),
            pl.BlockSpec((blk, 1), lambda i: (i, 0)),
            pl.BlockSpec((blk, 1), lambda i: (i, 0)),
            pl.BlockSpec((d_in, d_hid), lambda i: (0, 0)),
            pl.BlockSpec((1, d_hid), lambda i: (0, 0)),
            pl.BlockSpec((d_hid, d_out), lambda i: (0, 0)),
        ],
        out_specs=pl.BlockSpec((blk, d_out), lambda i: (i, 0)),
        out_shape=jax.ShapeDtypeStruct((n_pad, d_out), jnp.float32),
    )(agg1, ndst, nsrc, W1, b1_2d, W2)


def _tc_final(agg2, ndst, b2_2d, n_pad):
    blk = 1024
    d_out = b2_2d.shape[1]

    def body(q_ref, nd_ref, b2_ref, o_ref):
        o_ref[...] = (q_ref[0] + q_ref[1]) * nd_ref[...] + b2_ref[...]

    return pl.pallas_call(
        body,
        grid=(n_pad // blk,),
        in_specs=[
            pl.BlockSpec((NC, blk, d_out), lambda i: (0, i, 0)),
            pl.BlockSpec((blk, 1), lambda i: (i, 0)),
            pl.BlockSpec((1, d_out), lambda i: (0, 0)),
        ],
        out_specs=pl.BlockSpec((blk, d_out), lambda i: (i, 0)),
        out_shape=jax.ShapeDtypeStruct((n_pad, d_out), jnp.float32),
    )(agg2, ndst, b2_2d)


def kernel(x, edge_index, W1, b1, W2, b2):
    n = x.shape[0]
    e = edge_index.shape[1]

    nrows = -(-(n + 1) // LANES)
    if nrows % 8:
        nrows += 8 - nrows % 8
    n_pad = nrows * LANES
    k_chunks = -(-e // (NW * CHUNK))
    if k_chunks % NBUF:
        k_chunks += NBUF - k_chunks % NBUF
    epw = k_chunks * CHUNK
    e_pad = NW * epw

    src = edge_index[0].astype(jnp.int32)
    dst = edge_index[1].astype(jnp.int32)
    pad_idx = jnp.full((e_pad - e,), n, dtype=jnp.int32)
    src_p = jnp.concatenate([src, pad_idx])
    dst_p = jnp.concatenate([dst, pad_idx])
    src2 = src_p.reshape(NW, epw)
    dst2 = dst_p.reshape(NW, epw)
    edges3 = jnp.stack([src_p.reshape(NW, k_chunks, CHUNK),
                        dst_p.reshape(NW, k_chunks, CHUNK)], axis=2)

    zflat = jnp.zeros((n_pad,), dtype=jnp.float32)
    zeros_big = jnp.zeros((n_pad, LANES), dtype=jnp.float32)
    x_pad = jnp.pad(x, ((0, n_pad - n), (0, 0)))
    b1_2d = b1.reshape(1, -1)
    b2_2d = b2.reshape(1, -1)

    deg_raw = _degree_sc(src2, dst2, zflat, n_pad)

    norms = _tc_norms(deg_raw.reshape(NW, 2, nrows, LANES), nrows)
    norms_col = norms.reshape(2, n_pad, 1)
    nsrc = norms_col[0]
    ndst = norms_col[1]
    h0 = _tc_scale(x_pad, nsrc, n_pad)
    agg1 = _scatter_sc(h0, edges3, zeros_big, n_pad)
    g = _tc_mid(agg1, ndst, nsrc, W1, b1_2d, W2, n_pad)
    agg2 = _scatter_sc(g, edges3, zeros_big, n_pad)
    out = _tc_final(agg2, ndst, b2_2d, n_pad)
    return out[:n]


# speedup vs baseline: 10.4569x; 2.0634x over previous
import functools

import jax
import jax.numpy as jnp
from jax import lax
from jax.experimental import pallas as pl
from jax.experimental.pallas import tpu as pltpu
from jax.experimental.pallas import tpu_sc as plsc

LANES = 128
NC = 2
NS = 16
NW = NC * NS
CHUNK = 128
NBUF = 2
SIB = 4


def _sc_mesh():
    return plsc.VectorSubcoreMesh(
        core_axis_name="c", subcore_axis_name="s",
        num_cores=NC, num_subcores=NS)


_SC_PARAMS = pltpu.CompilerParams(needs_layout_passes=False)


def _degree_sc(src2, dst2, zflat, n_pad):
    epw = src2.shape[1]
    nsteps = epw // 16

    @functools.partial(
        pl.kernel,
        out_type=jax.ShapeDtypeStruct((NW, 2, n_pad), jnp.float32),
        mesh=_sc_mesh(),
        scratch_types=[
            pltpu.VMEM((epw,), jnp.int32),
            pltpu.VMEM((epw,), jnp.int32),
            pltpu.VMEM((n_pad,), jnp.float32),
            pltpu.VMEM((n_pad,), jnp.float32),
        ],
        compiler_params=_SC_PARAMS,
    )
    def deg_kernel(src_hbm, dst_hbm, zflat_hbm, out_hbm,
                   se_v, de_v, hist_o, hist_i):
        c = lax.axis_index("c")
        s = lax.axis_index("s")
        wid = c * NS + s
        pltpu.sync_copy(src_hbm.at[wid], se_v)
        pltpu.sync_copy(dst_hbm.at[wid], de_v)
        pltpu.sync_copy(zflat_hbm, hist_o)
        pltpu.sync_copy(zflat_hbm, hist_i)

        ones = jnp.full((16,), 1.0, dtype=jnp.float32)

        def step(i, carry):
            off = i * 16
            vs = se_v[pl.ds(off, 16)]
            plsc.addupdate_scatter(hist_o, [vs], ones)
            vd = de_v[pl.ds(off, 16)]
            plsc.addupdate_scatter(hist_i, [vd], ones)
            return carry

        lax.fori_loop(0, nsteps, step, 0)

        pltpu.sync_copy(hist_o, out_hbm.at[wid, 0])
        pltpu.sync_copy(hist_i, out_hbm.at[wid, 1])

    return deg_kernel(src2, dst2, zflat)


def _scatter_sc(h, edgesA, edgesB, zeros_big, n_pad):
    ka = edgesA.shape[1]
    kb = edgesB.shape[1]
    rpt = n_pad // NS

    @functools.partial(
        pl.kernel,
        out_type=jax.ShapeDtypeStruct((NC, n_pad, LANES), jnp.float32),
        mesh=_sc_mesh(),
        scratch_types=[
            pltpu.VMEM((SIB, 2, CHUNK), jnp.int32),
            pltpu.VMEM((NBUF, CHUNK, LANES), jnp.float32),
            pltpu.VMEM_SHARED((n_pad, LANES), jnp.float32),
            pltpu.SemaphoreType.DMA((SIB,)),
            pltpu.SemaphoreType.DMA((NBUF,)),
        ],
        compiler_params=_SC_PARAMS,
    )
    def scat_kernel(h_hbm, edgesA_hbm, edgesB_hbm, zeros_hbm, out_hbm,
                    idx_v, rows_v, acc_sh, isems, gsems):
        c = lax.axis_index("c")
        s = lax.axis_index("s")

        def run(edges_hbm, k_chunks):
            def idx_fetch(j, b):
                pltpu.async_copy(edges_hbm.at[s, j], idx_v.at[b],
                                 isems.at[b])

            def idx_wait(j, b):
                pltpu.make_async_copy(
                    edges_hbm.at[s, j], idx_v.at[b], isems.at[b]).wait()

            def gather(b, rb):
                pltpu.async_copy(h_hbm.at[idx_v.at[b, 0]], rows_v.at[rb],
                                 gsems.at[rb])

            def gather_wait(b, rb):
                pltpu.make_async_copy(h_hbm.at[idx_v.at[b, 0]],
                                      rows_v.at[rb], gsems.at[rb]).wait()

            def scat(b, rb):
                pltpu.sync_copy(rows_v.at[rb], acc_sh.at[idx_v.at[b, 1]],
                                add=True)

            for p in range(SIB):
                idx_fetch(p, p)
            idx_wait(0, 0)
            gather(0, 0)

            def step(j, carry):
                ib = lax.rem(j, SIB)
                pb = lax.rem(j - 1, SIB)
                rb = lax.rem(j, NBUF)
                prb = lax.rem(j - 1, NBUF)
                idx_wait(j, ib)
                gather(ib, rb)
                gather_wait(pb, prb)
                scat(pb, prb)
                @pl.when(j - 1 + SIB < k_chunks)
                def _():
                    idx_fetch(j - 1 + SIB, pb)
                return carry

            lax.fori_loop(1, k_chunks, step, 0, unroll=2)

            jl = k_chunks - 1
            lb = lax.rem(jl, SIB)
            lrb = lax.rem(jl, NBUF)
            gather_wait(lb, lrb)
            scat(lb, lrb)

        pltpu.sync_copy(zeros_hbm.at[pl.ds(s * rpt, rpt)],
                        acc_sh.at[pl.ds(s * rpt, rpt)])
        plsc.subcore_barrier()

        @pl.when(c == 0)
        def _():
            run(edgesA_hbm, ka)

        @pl.when(c == 1)
        def _():
            run(edgesB_hbm, kb)

        plsc.subcore_barrier()
        pltpu.sync_copy(acc_sh.at[pl.ds(s * rpt, rpt)],
                        out_hbm.at[c, pl.ds(s * rpt, rpt)])

    return scat_kernel(h, edgesA, edgesB, zeros_big)


def _tc_norms(deg_raw, nrows):
    def body(deg_ref, norm_ref):
        deg = jnp.sum(deg_ref[...], axis=0)
        norm_ref[...] = jnp.where(deg > 0, lax.rsqrt(deg), 0.0)

    return pl.pallas_call(
        body,
        out_shape=jax.ShapeDtypeStruct((2, nrows, LANES), jnp.float32),
    )(deg_raw)


def _tc_scale(x_pad, nsrc_col, n_pad):
    blk = 1024

    def body(x_ref, ns_ref, h0_ref):
        h0_ref[...] = x_ref[...] * ns_ref[...]

    return pl.pallas_call(
        body,
        grid=(n_pad // blk,),
        in_specs=[
            pl.BlockSpec((blk, LANES), lambda i: (i, 0)),
            pl.BlockSpec((blk, 1), lambda i: (i, 0)),
        ],
        out_specs=pl.BlockSpec((blk, LANES), lambda i: (i, 0)),
        out_shape=jax.ShapeDtypeStruct((n_pad, LANES), jnp.float32),
    )(x_pad, nsrc_col)


def _tc_mid(agg1, ndst, nsrc, W1, b1_2d, W2, n_pad):
    blk = 1024
    d_in, d_hid = W1.shape
    d_out = W2.shape[1]

    def body(p_ref, nd_ref, ns_ref, w1_ref, b1_ref, w2_ref, g_ref):
        a = (p_ref[0] + p_ref[1]) * nd_ref[...]
        h = jnp.dot(a, w1_ref[...], preferred_element_type=jnp.float32)
        h = jnp.maximum(h + b1_ref[...], 0.0)
        g_ref[...] = jnp.dot(h * ns_ref[...], w2_ref[...],
                             preferred_element_type=jnp.float32)

    return pl.pallas_call(
        body,
        grid=(n_pad // blk,),
        in_specs=[
            pl.BlockSpec((NC, blk, d_in), lambda i: (0, i, 0)),
            pl.BlockSpec((blk, 1), lambda i: (i, 0)),
            pl.BlockSpec((blk, 1), lambda i: (i, 0)),
            pl.BlockSpec((d_in, d_hid), lambda i: (0, 0)),
            pl.BlockSpec((1, d_hid), lambda i: (0, 0)),
            pl.BlockSpec((d_hid, d_out), lambda i: (0, 0)),
        ],
        out_specs=pl.BlockSpec((blk, d_out), lambda i: (i, 0)),
        out_shape=jax.ShapeDtypeStruct((n_pad, d_out), jnp.float32),
    )(agg1, ndst, nsrc, W1, b1_2d, W2)


def _tc_final(agg2, ndst, b2_2d, n_pad):
    blk = 1024
    d_out = b2_2d.shape[1]

    def body(q_ref, nd_ref, b2_ref, o_ref):
        o_ref[...] = (q_ref[0] + q_ref[1]) * nd_ref[...] + b2_ref[...]

    return pl.pallas_call(
        body,
        grid=(n_pad // blk,),
        in_specs=[
            pl.BlockSpec((NC, blk, d_out), lambda i: (0, i, 0)),
            pl.BlockSpec((blk, 1), lambda i: (i, 0)),
            pl.BlockSpec((1, d_out), lambda i: (0, 0)),
        ],
        out_specs=pl.BlockSpec((blk, d_out), lambda i: (i, 0)),
        out_shape=jax.ShapeDtypeStruct((n_pad, d_out), jnp.float32),
    )(agg2, ndst, b2_2d)


def kernel(x, edge_index, W1, b1, W2, b2):
    n = x.shape[0]
    e = edge_index.shape[1]

    nrows = -(-(n + 1) // LANES)
    if nrows % 8:
        nrows += 8 - nrows % 8
    n_pad = nrows * LANES
    t_chunks = -(-e // (NS * CHUNK))
    ka = max(SIB, min(t_chunks - SIB, round(t_chunks * 0.77)))
    kb = t_chunks - ka
    e_pad = NS * t_chunks * CHUNK
    epw = e_pad // NW

    src = edge_index[0].astype(jnp.int32)
    dst = edge_index[1].astype(jnp.int32)
    pad_idx = jnp.full((e_pad - e,), n, dtype=jnp.int32)
    src_p = jnp.concatenate([src, pad_idx])
    dst_p = jnp.concatenate([dst, pad_idx])
    src2 = src_p.reshape(NW, epw)
    dst2 = dst_p.reshape(NW, epw)
    ea = NS * ka * CHUNK
    edgesA = jnp.stack([src_p[:ea].reshape(NS, ka, CHUNK),
                        dst_p[:ea].reshape(NS, ka, CHUNK)], axis=2)
    edgesB = jnp.stack([src_p[ea:].reshape(NS, kb, CHUNK),
                        dst_p[ea:].reshape(NS, kb, CHUNK)], axis=2)

    zflat = jnp.zeros((n_pad,), dtype=jnp.float32)
    zeros_big = jnp.zeros((n_pad, LANES), dtype=jnp.float32)
    x_pad = jnp.pad(x, ((0, n_pad - n), (0, 0)))
    b1_2d = b1.reshape(1, -1)
    b2_2d = b2.reshape(1, -1)

    deg_raw = _degree_sc(src2, dst2, zflat, n_pad)

    norms = _tc_norms(deg_raw.reshape(NW, 2, nrows, LANES), nrows)
    norms_col = norms.reshape(2, n_pad, 1)
    nsrc = norms_col[0]
    ndst = norms_col[1]
    h0 = _tc_scale(x_pad, nsrc, n_pad)
    agg1 = _scatter_sc(h0, edgesA, edgesB, zeros_big, n_pad)
    g = _tc_mid(agg1, ndst, nsrc, W1, b1_2d, W2, n_pad)
    agg2 = _scatter_sc(g, edgesA, edgesB, zeros_big, n_pad)
    out = _tc_final(agg2, ndst, b2_2d, n_pad)
    return out[:n]


# speedup vs baseline: 11.0162x; 1.0535x over previous
import functools

import jax
import jax.numpy as jnp
from jax import lax
from jax.experimental import pallas as pl
from jax.experimental.pallas import tpu as pltpu
from jax.experimental.pallas import tpu_sc as plsc

LANES = 128
NC = 2
NS = 16
NW = NC * NS
CHUNK = 128
NBUF = 2
SIB = 4


def _sc_mesh():
    return plsc.VectorSubcoreMesh(
        core_axis_name="c", subcore_axis_name="s",
        num_cores=NC, num_subcores=NS)


_SC_PARAMS = pltpu.CompilerParams(needs_layout_passes=False)


def _degree_sc(src2, dst2, zflat, n_pad):
    epw = src2.shape[1]
    nsteps = epw // 16

    @functools.partial(
        pl.kernel,
        out_type=jax.ShapeDtypeStruct((NW, 2, n_pad), jnp.float32),
        mesh=_sc_mesh(),
        scratch_types=[
            pltpu.VMEM((epw,), jnp.int32),
            pltpu.VMEM((epw,), jnp.int32),
            pltpu.VMEM((n_pad,), jnp.float32),
            pltpu.VMEM((n_pad,), jnp.float32),
        ],
        compiler_params=_SC_PARAMS,
    )
    def deg_kernel(src_hbm, dst_hbm, zflat_hbm, out_hbm,
                   se_v, de_v, hist_o, hist_i):
        c = lax.axis_index("c")
        s = lax.axis_index("s")
        wid = c * NS + s
        pltpu.sync_copy(src_hbm.at[wid], se_v)
        pltpu.sync_copy(dst_hbm.at[wid], de_v)
        pltpu.sync_copy(zflat_hbm, hist_o)
        pltpu.sync_copy(zflat_hbm, hist_i)

        ones = jnp.full((16,), 1.0, dtype=jnp.float32)

        def step(i, carry):
            off = i * 16
            vs = se_v[pl.ds(off, 16)]
            plsc.addupdate_scatter(hist_o, [vs], ones)
            vd = de_v[pl.ds(off, 16)]
            plsc.addupdate_scatter(hist_i, [vd], ones)
            return carry

        lax.fori_loop(0, nsteps, step, 0)

        pltpu.sync_copy(hist_o, out_hbm.at[wid, 0])
        pltpu.sync_copy(hist_i, out_hbm.at[wid, 1])

    return deg_kernel(src2, dst2, zflat)


def _scatter_sc(h, edgesA, edgesB, zeros_big, n_pad):
    ka = edgesA.shape[1]
    kb = edgesB.shape[1]
    rpt = n_pad // NS

    @functools.partial(
        pl.kernel,
        out_type=jax.ShapeDtypeStruct((NC, n_pad, LANES), jnp.float32),
        mesh=_sc_mesh(),
        scratch_types=[
            pltpu.VMEM((SIB, 2, CHUNK), jnp.int32),
            pltpu.VMEM((NBUF, CHUNK, LANES), jnp.float32),
            pltpu.VMEM_SHARED((n_pad, LANES), jnp.float32),
            pltpu.SemaphoreType.DMA((SIB,)),
            pltpu.SemaphoreType.DMA((NBUF,)),
        ],
        compiler_params=_SC_PARAMS,
    )
    def scat_kernel(h_hbm, edgesA_hbm, edgesB_hbm, zeros_hbm, out_hbm,
                    idx_v, rows_v, acc_sh, isems, gsems):
        c = lax.axis_index("c")
        s = lax.axis_index("s")

        def run(edges_hbm, k_chunks):
            def idx_fetch(j, b):
                pltpu.async_copy(edges_hbm.at[s, j], idx_v.at[b],
                                 isems.at[b])

            def idx_wait(j, b):
                pltpu.make_async_copy(
                    edges_hbm.at[s, j], idx_v.at[b], isems.at[b]).wait()

            def gather(b, rb):
                pltpu.async_copy(h_hbm.at[idx_v.at[b, 0]], rows_v.at[rb],
                                 gsems.at[rb])

            def gather_wait(b, rb):
                pltpu.make_async_copy(h_hbm.at[idx_v.at[b, 0]],
                                      rows_v.at[rb], gsems.at[rb]).wait()

            def scat(b, rb):
                pltpu.sync_copy(rows_v.at[rb], acc_sh.at[idx_v.at[b, 1]],
                                add=True)

            for p in range(SIB):
                idx_fetch(p, p)
            idx_wait(0, 0)
            gather(0, 0)

            def step(j, carry):
                ib = lax.rem(j, SIB)
                pb = lax.rem(j - 1, SIB)
                rb = lax.rem(j, NBUF)
                prb = lax.rem(j - 1, NBUF)
                idx_wait(j, ib)
                gather(ib, rb)
                gather_wait(pb, prb)
                scat(pb, prb)
                @pl.when(j - 1 + SIB < k_chunks)
                def _():
                    idx_fetch(j - 1 + SIB, pb)
                return carry

            lax.fori_loop(1, k_chunks, step, 0, unroll=2)

            jl = k_chunks - 1
            lb = lax.rem(jl, SIB)
            lrb = lax.rem(jl, NBUF)
            gather_wait(lb, lrb)
            scat(lb, lrb)

        pltpu.sync_copy(zeros_hbm.at[pl.ds(s * rpt, rpt)],
                        acc_sh.at[pl.ds(s * rpt, rpt)])
        plsc.subcore_barrier()

        @pl.when(c == 0)
        def _():
            run(edgesA_hbm, ka)

        @pl.when(c == 1)
        def _():
            run(edgesB_hbm, kb)

        plsc.subcore_barrier()
        pltpu.sync_copy(acc_sh.at[pl.ds(s * rpt, rpt)],
                        out_hbm.at[c, pl.ds(s * rpt, rpt)])

    return scat_kernel(h, edgesA, edgesB, zeros_big)


def _tc_norms(deg_raw, nrows):
    def body(deg_ref, norm_ref):
        deg = jnp.sum(deg_ref[...], axis=0)
        norm_ref[...] = jnp.where(deg > 0, lax.rsqrt(deg), 0.0)

    return pl.pallas_call(
        body,
        out_shape=jax.ShapeDtypeStruct((2, nrows, LANES), jnp.float32),
    )(deg_raw)


def _tc_scale(x, nsrc_col, blk):
    n = x.shape[0]

    def body(x_ref, ns_ref, h0_ref):
        h0_ref[...] = x_ref[...] * ns_ref[...]

    return pl.pallas_call(
        body,
        grid=(n // blk,),
        in_specs=[
            pl.BlockSpec((blk, LANES), lambda i: (i, 0)),
            pl.BlockSpec((blk, 1), lambda i: (i, 0)),
        ],
        out_specs=pl.BlockSpec((blk, LANES), lambda i: (i, 0)),
        out_shape=jax.ShapeDtypeStruct((n, LANES), jnp.float32),
    )(x, nsrc_col)


def _tc_mid(agg1, ndst, nsrc, W1, b1_2d, W2, n, blk):
    d_in, d_hid = W1.shape
    d_out = W2.shape[1]

    def body(p_ref, nd_ref, ns_ref, w1_ref, b1_ref, w2_ref, g_ref):
        a = (p_ref[0] + p_ref[1]) * nd_ref[...]
        h = jnp.dot(a, w1_ref[...], preferred_element_type=jnp.float32)
        h = jnp.maximum(h + b1_ref[...], 0.0)
        g_ref[...] = jnp.dot(h * ns_ref[...], w2_ref[...],
                             preferred_element_type=jnp.float32)

    return pl.pallas_call(
        body,
        grid=(n // blk,),
        in_specs=[
            pl.BlockSpec((NC, blk, d_in), lambda i: (0, i, 0)),
            pl.BlockSpec((blk, 1), lambda i: (i, 0)),
            pl.BlockSpec((blk, 1), lambda i: (i, 0)),
            pl.BlockSpec((d_in, d_hid), lambda i: (0, 0)),
            pl.BlockSpec((1, d_hid), lambda i: (0, 0)),
            pl.BlockSpec((d_hid, d_out), lambda i: (0, 0)),
        ],
        out_specs=pl.BlockSpec((blk, d_out), lambda i: (i, 0)),
        out_shape=jax.ShapeDtypeStruct((n, d_out), jnp.float32),
    )(agg1, ndst, nsrc, W1, b1_2d, W2)


def _tc_final(agg2, ndst, b2_2d, n, blk):
    d_out = b2_2d.shape[1]

    def body(q_ref, nd_ref, b2_ref, o_ref):
        o_ref[...] = (q_ref[0] + q_ref[1]) * nd_ref[...] + b2_ref[...]

    return pl.pallas_call(
        body,
        grid=(n // blk,),
        in_specs=[
            pl.BlockSpec((NC, blk, d_out), lambda i: (0, i, 0)),
            pl.BlockSpec((blk, 1), lambda i: (i, 0)),
            pl.BlockSpec((1, d_out), lambda i: (0, 0)),
        ],
        out_specs=pl.BlockSpec((blk, d_out), lambda i: (i, 0)),
        out_shape=jax.ShapeDtypeStruct((n, d_out), jnp.float32),
    )(agg2, ndst, b2_2d)


def kernel(x, edge_index, W1, b1, W2, b2):
    n = x.shape[0]
    e = edge_index.shape[1]

    nrows = -(-(n + 1) // LANES)
    if nrows % 8:
        nrows += 8 - nrows % 8
    n_pad = nrows * LANES
    t_chunks = -(-e // (NS * CHUNK))
    ka = max(SIB, min(t_chunks - SIB, round(t_chunks * 0.73)))
    kb = t_chunks - ka
    e_pad = NS * t_chunks * CHUNK
    epw = e_pad // NW

    blk = 8
    for cand in (1024, 1000, 800, 640, 512, 400, 256, 200, 128, 80, 40, 16):
        if n % cand == 0:
            blk = cand
            break

    src = edge_index[0].astype(jnp.int32)
    dst = edge_index[1].astype(jnp.int32)
    pad_idx = jnp.full((e_pad - e,), n, dtype=jnp.int32)
    src2 = jnp.concatenate([src, pad_idx]).reshape(NW, epw)
    dst2 = jnp.concatenate([dst, pad_idx]).reshape(NW, epw)
    src_p = jnp.concatenate([src, jnp.zeros((e_pad - e,), jnp.int32)])
    dst_p = jnp.concatenate([dst, pad_idx])
    ea = NS * ka * CHUNK
    edgesA = jnp.stack([src_p[:ea].reshape(NS, ka, CHUNK),
                        dst_p[:ea].reshape(NS, ka, CHUNK)], axis=2)
    edgesB = jnp.stack([src_p[ea:].reshape(NS, kb, CHUNK),
                        dst_p[ea:].reshape(NS, kb, CHUNK)], axis=2)

    zflat = jnp.zeros((n_pad,), dtype=jnp.float32)
    zeros_big = jnp.zeros((n_pad, LANES), dtype=jnp.float32)
    b1_2d = b1.reshape(1, -1)
    b2_2d = b2.reshape(1, -1)

    deg_raw = _degree_sc(src2, dst2, zflat, n_pad)

    norms_col = _tc_norms(deg_raw.reshape(NW, 2, nrows, LANES),
                          nrows).reshape(2, n_pad, 1)
    nsrc = norms_col[0]
    ndst = norms_col[1]
    h0 = _tc_scale(x, nsrc, blk)
    agg1 = _scatter_sc(h0, edgesA, edgesB, zeros_big, n_pad)
    g = _tc_mid(agg1, ndst, nsrc, W1, b1_2d, W2, n, blk)
    agg2 = _scatter_sc(g, edgesA, edgesB, zeros_big, n_pad)
    return _tc_final(agg2, ndst, b2_2d, n, blk)


# speedup vs baseline: 11.5677x; 1.0501x over previous
import functools

import jax
import jax.numpy as jnp
from jax import lax
from jax.experimental import pallas as pl
from jax.experimental.pallas import tpu as pltpu
from jax.experimental.pallas import tpu_sc as plsc

LANES = 128
NC = 2
NS = 16
NW = NC * NS
CHUNK = 128
NBUF = 3
SIB = 5


def _sc_mesh():
    return plsc.VectorSubcoreMesh(
        core_axis_name="c", subcore_axis_name="s",
        num_cores=NC, num_subcores=NS)


_SC_PARAMS = pltpu.CompilerParams(needs_layout_passes=False)


def _degree_sc(src2, dst2, zflat, n_pad):
    epw = src2.shape[1]
    nsteps = epw // 16

    @functools.partial(
        pl.kernel,
        out_type=jax.ShapeDtypeStruct((NW, 2, n_pad), jnp.float32),
        mesh=_sc_mesh(),
        scratch_types=[
            pltpu.VMEM((epw,), jnp.int32),
            pltpu.VMEM((epw,), jnp.int32),
            pltpu.VMEM((n_pad,), jnp.float32),
            pltpu.VMEM((n_pad,), jnp.float32),
        ],
        compiler_params=_SC_PARAMS,
    )
    def deg_kernel(src_hbm, dst_hbm, zflat_hbm, out_hbm,
                   se_v, de_v, hist_o, hist_i):
        c = lax.axis_index("c")
        s = lax.axis_index("s")
        wid = c * NS + s
        pltpu.sync_copy(src_hbm.at[wid], se_v)
        pltpu.sync_copy(dst_hbm.at[wid], de_v)
        pltpu.sync_copy(zflat_hbm, hist_o)
        pltpu.sync_copy(zflat_hbm, hist_i)

        ones = jnp.full((16,), 1.0, dtype=jnp.float32)

        def step(i, carry):
            off = i * 16
            vs = se_v[pl.ds(off, 16)]
            plsc.addupdate_scatter(hist_o, [vs], ones)
            vd = de_v[pl.ds(off, 16)]
            plsc.addupdate_scatter(hist_i, [vd], ones)
            return carry

        lax.fori_loop(0, nsteps, step, 0)

        pltpu.sync_copy(hist_o, out_hbm.at[wid, 0])
        pltpu.sync_copy(hist_i, out_hbm.at[wid, 1])

    return deg_kernel(src2, dst2, zflat)


def _scatter_sc(h, edgesA, edgesB, zeros_big, acc_rows):
    ka = edgesA.shape[1]
    kb = edgesB.shape[1]
    stride = (-(-acc_rows // NS) + 7) // 8 * 8
    last = acc_rows - (NS - 1) * stride
    assert 0 < last <= stride

    @functools.partial(
        pl.kernel,
        out_type=jax.ShapeDtypeStruct((NC, acc_rows, LANES), jnp.float32),
        mesh=_sc_mesh(),
        scratch_types=[
            pltpu.VMEM((SIB, 2, CHUNK), jnp.int32),
            pltpu.VMEM((NBUF, CHUNK, LANES), jnp.float32),
            pltpu.VMEM_SHARED((acc_rows, LANES), jnp.float32),
            pltpu.SemaphoreType.DMA((SIB,)),
            pltpu.SemaphoreType.DMA((NBUF,)),
            pltpu.SemaphoreType.DMA((NBUF,)),
        ],
        compiler_params=_SC_PARAMS,
    )
    def scat_kernel(h_hbm, edgesA_hbm, edgesB_hbm, zeros_hbm, out_hbm,
                    idx_v, rows_v, acc_sh, isems, gsems, ssems):
        c = lax.axis_index("c")
        s = lax.axis_index("s")

        def run(edges_hbm, k_chunks):
            def idx_fetch(j, b):
                pltpu.async_copy(edges_hbm.at[s, j], idx_v.at[b],
                                 isems.at[b])

            def idx_wait(j, b):
                pltpu.make_async_copy(
                    edges_hbm.at[s, j], idx_v.at[b], isems.at[b]).wait()

            def gather(b, rb):
                pltpu.async_copy(h_hbm.at[idx_v.at[b, 0]], rows_v.at[rb],
                                 gsems.at[rb])

            def gather_wait(b, rb):
                pltpu.make_async_copy(h_hbm.at[idx_v.at[b, 0]],
                                      rows_v.at[rb], gsems.at[rb]).wait()

            def scat(b, rb):
                pltpu.async_copy(rows_v.at[rb], acc_sh.at[idx_v.at[b, 1]],
                                 ssems.at[rb], add=True)

            def scat_wait(b, rb):
                pltpu.make_async_copy(rows_v.at[rb],
                                      acc_sh.at[idx_v.at[b, 1]],
                                      ssems.at[rb]).wait()

            for p in range(SIB):
                idx_fetch(p, p)

            def step(j, carry):
                ib = lax.rem(j, SIB)
                rb = lax.rem(j, NBUF)

                @pl.when(j >= NBUF)
                def _():
                    fb = lax.rem(j - NBUF, SIB)
                    scat_wait(fb, rb)

                    @pl.when(j - NBUF + SIB < k_chunks)
                    def _():
                        idx_fetch(j - NBUF + SIB, fb)

                idx_wait(j, ib)
                gather(ib, rb)

                @pl.when(j >= 1)
                def _():
                    pb = lax.rem(j - 1, SIB)
                    prb = lax.rem(j - 1, NBUF)
                    gather_wait(pb, prb)
                    scat(pb, prb)

                return carry

            lax.fori_loop(0, k_chunks, step, 0, unroll=2)

            jl = k_chunks - 1
            gather_wait(lax.rem(jl, SIB), lax.rem(jl, NBUF))
            scat(lax.rem(jl, SIB), lax.rem(jl, NBUF))
            for t in range(NBUF):
                jt = k_chunks - NBUF + t
                scat_wait(lax.rem(jt, SIB), lax.rem(jt, NBUF))

        @pl.when(s < NS - 1)
        def _():
            pltpu.sync_copy(zeros_hbm.at[pl.ds(s * stride, stride)],
                            acc_sh.at[pl.ds(s * stride, stride)])

        @pl.when(s == NS - 1)
        def _():
            pltpu.sync_copy(zeros_hbm.at[pl.ds((NS - 1) * stride, last)],
                            acc_sh.at[pl.ds((NS - 1) * stride, last)])

        plsc.subcore_barrier()

        @pl.when(c == 0)
        def _():
            run(edgesA_hbm, ka)

        @pl.when(c == 1)
        def _():
            run(edgesB_hbm, kb)

        plsc.subcore_barrier()

        @pl.when(s < NS - 1)
        def _():
            pltpu.sync_copy(acc_sh.at[pl.ds(s * stride, stride)],
                            out_hbm.at[c, pl.ds(s * stride, stride)])

        @pl.when(s == NS - 1)
        def _():
            pltpu.sync_copy(acc_sh.at[pl.ds((NS - 1) * stride, last)],
                            out_hbm.at[c, pl.ds((NS - 1) * stride, last)])

    return scat_kernel(h, edgesA, edgesB, zeros_big)


def _tc_norms(deg_raw, nrows):
    def body(deg_ref, norm_ref):
        deg = jnp.sum(deg_ref[...], axis=0)
        norm_ref[...] = jnp.where(deg > 0, lax.rsqrt(deg), 0.0)

    return pl.pallas_call(
        body,
        out_shape=jax.ShapeDtypeStruct((2, nrows, LANES), jnp.float32),
    )(deg_raw)


def _tc_scale(x, nsrc_col, blk):
    n = x.shape[0]

    def body(x_ref, ns_ref, h0_ref):
        h0_ref[...] = x_ref[...] * ns_ref[...]

    return pl.pallas_call(
        body,
        grid=(n // blk,),
        in_specs=[
            pl.BlockSpec((blk, LANES), lambda i: (i, 0)),
            pl.BlockSpec((blk, 1), lambda i: (i, 0)),
        ],
        out_specs=pl.BlockSpec((blk, LANES), lambda i: (i, 0)),
        out_shape=jax.ShapeDtypeStruct((n, LANES), jnp.float32),
    )(x, nsrc_col)


def _tc_mid(agg1, ndst, nsrc, W1, b1_2d, W2, n, blk):
    d_in, d_hid = W1.shape
    d_out = W2.shape[1]

    def body(p_ref, nd_ref, ns_ref, w1_ref, b1_ref, w2_ref, g_ref):
        a = (p_ref[0] + p_ref[1]) * nd_ref[...]
        h = jnp.dot(a, w1_ref[...], preferred_element_type=jnp.float32)
        h = jnp.maximum(h + b1_ref[...], 0.0)
        g_ref[...] = jnp.dot(h * ns_ref[...], w2_ref[...],
                             preferred_element_type=jnp.float32)

    return pl.pallas_call(
        body,
        grid=(n // blk,),
        in_specs=[
            pl.BlockSpec((NC, blk, d_in), lambda i: (0, i, 0)),
            pl.BlockSpec((blk, 1), lambda i: (i, 0)),
            pl.BlockSpec((blk, 1), lambda i: (i, 0)),
            pl.BlockSpec((d_in, d_hid), lambda i: (0, 0)),
            pl.BlockSpec((1, d_hid), lambda i: (0, 0)),
            pl.BlockSpec((d_hid, d_out), lambda i: (0, 0)),
        ],
        out_specs=pl.BlockSpec((blk, d_out), lambda i: (i, 0)),
        out_shape=jax.ShapeDtypeStruct((n, d_out), jnp.float32),
    )(agg1, ndst, nsrc, W1, b1_2d, W2)


def _tc_final(agg2, ndst, b2_2d, n, blk):
    d_out = b2_2d.shape[1]

    def body(q_ref, nd_ref, b2_ref, o_ref):
        o_ref[...] = (q_ref[0] + q_ref[1]) * nd_ref[...] + b2_ref[...]

    return pl.pallas_call(
        body,
        grid=(n // blk,),
        in_specs=[
            pl.BlockSpec((NC, blk, d_out), lambda i: (0, i, 0)),
            pl.BlockSpec((blk, 1), lambda i: (i, 0)),
            pl.BlockSpec((1, d_out), lambda i: (0, 0)),
        ],
        out_specs=pl.BlockSpec((blk, d_out), lambda i: (i, 0)),
        out_shape=jax.ShapeDtypeStruct((n, d_out), jnp.float32),
    )(agg2, ndst, b2_2d)


def kernel(x, edge_index, W1, b1, W2, b2):
    n = x.shape[0]
    e = edge_index.shape[1]

    nrows = -(-(n + 1) // LANES)
    if nrows % 8:
        nrows += 8 - nrows % 8
    n_pad = nrows * LANES
    t_chunks = -(-e // (NS * CHUNK))
    ka = max(SIB, min(t_chunks - SIB, round(t_chunks * 0.73)))
    kb = t_chunks - ka
    e_pad = NS * t_chunks * CHUNK
    epw = e_pad // NW

    blk = 8
    for cand in (1024, 1000, 800, 640, 512, 400, 256, 200, 128, 80, 40, 16):
        if n % cand == 0:
            blk = cand
            break

    src = edge_index[0].astype(jnp.int32)
    dst = edge_index[1].astype(jnp.int32)
    pad_idx = jnp.full((e_pad - e,), n, dtype=jnp.int32)
    src2 = jnp.concatenate([src, pad_idx]).reshape(NW, epw)
    dst2 = jnp.concatenate([dst, pad_idx]).reshape(NW, epw)
    src_p = jnp.concatenate([src, jnp.zeros((e_pad - e,), jnp.int32)])
    dst_p = jnp.concatenate([dst, pad_idx])
    ea = NS * ka * CHUNK
    edgesA = jnp.stack([src_p[:ea].reshape(NS, ka, CHUNK),
                        dst_p[:ea].reshape(NS, ka, CHUNK)], axis=2)
    edgesB = jnp.stack([src_p[ea:].reshape(NS, kb, CHUNK),
                        dst_p[ea:].reshape(NS, kb, CHUNK)], axis=2)

    acc_rows = -(-(n + 1) // NS) * NS
    zflat = jnp.zeros((n_pad,), dtype=jnp.float32)
    zeros_big = jnp.zeros((acc_rows, LANES), dtype=jnp.float32)
    b1_2d = b1.reshape(1, -1)
    b2_2d = b2.reshape(1, -1)

    deg_raw = _degree_sc(src2, dst2, zflat, n_pad)

    norms_col = _tc_norms(deg_raw.reshape(NW, 2, nrows, LANES),
                          nrows).reshape(2, n_pad, 1)
    nsrc = norms_col[0]
    ndst = norms_col[1]
    h0 = _tc_scale(x, nsrc, blk)
    agg1 = _scatter_sc(h0, edgesA, edgesB, zeros_big, acc_rows)
    g = _tc_mid(agg1, ndst, nsrc, W1, b1_2d, W2, n, blk)
    agg2 = _scatter_sc(g, edgesA, edgesB, zeros_big, acc_rows)
    return _tc_final(agg2, ndst, b2_2d, n, blk)
